# baseline (speedup 1.0000x reference)
import functools

import jax
import jax.numpy as jnp
from jax import lax
from jax.experimental import pallas as pl
from jax.experimental.pallas import tpu as pltpu
from jax.experimental.pallas import tpu_sc as plsc

B = 16384
D = 64
NC = 2
NS = 16
NW = NC * NS
BPW = B // NW
HB = BPW // 2
L = 16


def _sc_kernel(ent_idx_hbm, rel_idx_hbm, ts_hbm, ent_tab, rel_tab, time_tab,
               decay_hbm, out_ent, out_comb,
               eidx_v, ridx_v, ts_v, fac_v, dec_v, ent_v, time_v, rel_v,
               sem_e, sem_rt):
    wid = lax.axis_index("s") * NC + lax.axis_index("c")
    base = wid * BPW

    pltpu.sync_copy(ent_idx_hbm.at[pl.ds(base, BPW)], eidx_v)
    pltpu.sync_copy(rel_idx_hbm.at[pl.ds(base, BPW)], ridx_v)
    pltpu.sync_copy(ts_hbm.at[pl.ds(base, BPW)], ts_v)
    pltpu.sync_copy(decay_hbm, dec_v)

    def ent_issue(half):
        def issue(k, carry):
            off = pl.multiple_of(half * HB + k * L, L)
            vec = eidx_v[pl.ds(off, L)]
            dst = pl.multiple_of(k * L, L)
            for l in range(L):
                r = vec[l]
                pltpu.async_copy(ent_tab.at[pl.ds(r, 1), :],
                                 ent_v.at[pl.ds(dst + l, 1), :], sem_e)
            return carry

        lax.fori_loop(0, HB // L, issue, 0)

    def rt_issue(half):
        def issue(k, carry):
            off = pl.multiple_of(half * HB + k * L, L)
            vec = ridx_v[pl.ds(off, L)]
            dst = pl.multiple_of(k * L, L)
            for l in range(L):
                r = vec[l]
                pltpu.async_copy(time_tab.at[pl.ds(r, 1), :],
                                 time_v.at[pl.ds(dst + l, 1), :], sem_rt)
                pltpu.async_copy(rel_tab.at[pl.ds(r, 1), :],
                                 rel_v.at[pl.ds(dst + l, 1), :], sem_rt)
            return carry

        lax.fori_loop(0, HB // L, issue, 0)

    def ent_drain_write(half):
        pltpu.make_async_copy(ent_tab.at[pl.ds(0, HB), :], ent_v, sem_e).wait()
        pltpu.sync_copy(ent_v, out_ent.at[pl.ds(base + half * HB, HB), :])

    def rt_drain(half):
        pltpu.make_async_copy(ent_tab.at[pl.ds(0, HB), :], time_v,
                              sem_rt).wait()
        pltpu.make_async_copy(ent_tab.at[pl.ds(0, HB), :], rel_v,
                              sem_rt).wait()

    def combine_write(half):
        def comb_body(i, carry):
            fi = jnp.full((L,), half * HB, jnp.int32) + i
            f = plsc.load_gather(fac_v, [fi])
            for c in range(D // L):
                sl = pl.ds(c * L, L)
                time_v[i, sl] = time_v[i, sl] * f + rel_v[i, sl]
            return carry

        lax.fori_loop(0, HB, comb_body, 0)
        pltpu.sync_copy(time_v, out_comb.at[pl.ds(base + half * HB, HB), :])

    ent_issue(0)
    rt_issue(0)

    neg_decay = dec_v[...]

    def fac_body(k, carry):
        off = pl.multiple_of(k * L, L)
        t = ts_v[pl.ds(off, L)].astype(jnp.float32)
        fac_v[pl.ds(off, L)] = jnp.exp(neg_decay * t)
        return carry

    lax.fori_loop(0, BPW // L, fac_body, 0)

    ent_drain_write(0)
    ent_issue(1)
    rt_drain(0)
    combine_write(0)
    rt_issue(1)
    ent_drain_write(1)
    rt_drain(1)
    combine_write(1)


@jax.jit
def _run(entity_indices, relation_indices, timestamp_indices, entity_table,
         relation_table, time_table, neg_decay):
    mesh = plsc.VectorSubcoreMesh(core_axis_name="c", subcore_axis_name="s",
                                  num_cores=NC, num_subcores=NS)
    f = functools.partial(
        pl.kernel,
        out_type=[
            jax.ShapeDtypeStruct((B, D), jnp.float32),
            jax.ShapeDtypeStruct((B, D), jnp.float32),
        ],
        mesh=mesh,
        scratch_types=[
            pltpu.VMEM((BPW,), jnp.int32),
            pltpu.VMEM((BPW,), jnp.int32),
            pltpu.VMEM((BPW,), jnp.int32),
            pltpu.VMEM((BPW,), jnp.float32),
            pltpu.VMEM((L,), jnp.float32),
            pltpu.VMEM((HB, D), jnp.float32),
            pltpu.VMEM((HB, D), jnp.float32),
            pltpu.VMEM((HB, D), jnp.float32),
            pltpu.SemaphoreType.DMA,
            pltpu.SemaphoreType.DMA,
        ],
        compiler_params=pltpu.CompilerParams(needs_layout_passes=False,
                                             use_tc_tiling_on_sc=True),
        name="kg_embeddings_sc",
    )(_sc_kernel)
    return f(entity_indices, relation_indices, timestamp_indices,
             entity_table, relation_table, time_table, neg_decay)


def kernel(entity_indices, relation_indices, timestamp_indices, entity_table,
           relation_table, time_table, time_decay):
    ent_idx = entity_indices.astype(jnp.int32)
    rel_idx = relation_indices.astype(jnp.int32)
    ts = timestamp_indices.astype(jnp.int32)
    neg_decay = jnp.broadcast_to(-time_decay, (L,))
    out_ent, out_comb = _run(ent_idx, rel_idx, ts, entity_table,
                             relation_table, time_table, neg_decay)
    decoy = jnp.take(entity_table, rel_idx, axis=0)
    zero = jax.lax.optimization_barrier(jnp.float32(0.0))
    out_ent = out_ent + zero * decoy
    return (out_ent, out_comb)


# speedup vs baseline: 1.0096x; 1.0096x over previous
import functools

import jax
import jax.numpy as jnp
from jax import lax
from jax.experimental import pallas as pl
from jax.experimental.pallas import tpu as pltpu
from jax.experimental.pallas import tpu_sc as plsc

B = 16384
D = 64
NC = 2
NS = 16
NW = NC * NS
BPW = B // NW
HB = BPW // 2
L = 16

_PARAMS = pltpu.CompilerParams(needs_layout_passes=False,
                               use_tc_tiling_on_sc=True)


def _ent_kernel(ent_idx_hbm, ent_tab, outT_ent, eidx_v, ent_v, entT_v, sem):
    wid = lax.axis_index("s") * NC + lax.axis_index("c")
    base = wid * BPW
    pltpu.sync_copy(ent_idx_hbm.at[pl.ds(base, BPW)], eidx_v)

    def issue(k, carry):
        off = pl.multiple_of(k * L, L)
        vec = eidx_v[pl.ds(off, L)]
        for l in range(L):
            r = vec[l]
            pltpu.async_copy(ent_tab.at[pl.ds(r, 1), :],
                             ent_v.at[pl.ds(off + l, 1), :], sem)
        return carry

    lax.fori_loop(0, BPW // L, issue, 0)

    pltpu.make_async_copy(ent_tab.at[pl.ds(0, BPW), :], ent_v, sem).wait()

    jvecs = [lax.iota(jnp.int32, L) + c * L for c in range(D // L)]

    def transpose_body(i, carry):
        ivec = jnp.full((L,), i, jnp.int32)
        for c in range(D // L):
            v = ent_v[i, pl.ds(c * L, L)]
            plsc.store_scatter(entT_v, [jvecs[c], ivec], v)
        return carry

    lax.fori_loop(0, BPW, transpose_body, 0)
    pltpu.sync_copy(entT_v, outT_ent.at[:, pl.ds(base, BPW)])


def _comb_kernel(rel_idx_hbm, ts_hbm, rel_tab, time_tab, decay_hbm,
                 outT_comb, ridx_v, ts_v, fac_v, dec_v, time_v, rel_v,
                 combT_v, sem):
    wid = lax.axis_index("s") * NC + lax.axis_index("c")
    base = wid * BPW
    pltpu.sync_copy(rel_idx_hbm.at[pl.ds(base, BPW)], ridx_v)
    pltpu.sync_copy(ts_hbm.at[pl.ds(base, BPW)], ts_v)
    pltpu.sync_copy(decay_hbm, dec_v)

    def rt_issue(half):
        def issue(k, carry):
            off = pl.multiple_of(half * HB + k * L, L)
            vec = ridx_v[pl.ds(off, L)]
            dst = pl.multiple_of(k * L, L)
            for l in range(L):
                r = vec[l]
                pltpu.async_copy(time_tab.at[pl.ds(r, 1), :],
                                 time_v.at[pl.ds(dst + l, 1), :], sem)
                pltpu.async_copy(rel_tab.at[pl.ds(r, 1), :],
                                 rel_v.at[pl.ds(dst + l, 1), :], sem)
            return carry

        lax.fori_loop(0, HB // L, issue, 0)

    rt_issue(0)

    neg_decay = dec_v[...]

    def fac_body(k, carry):
        off = pl.multiple_of(k * L, L)
        t = ts_v[pl.ds(off, L)].astype(jnp.float32)
        fac_v[pl.ds(off, L)] = jnp.exp(neg_decay * t)
        return carry

    lax.fori_loop(0, BPW // L, fac_body, 0)

    jvecs = [lax.iota(jnp.int32, L) + c * L for c in range(D // L)]

    def rt_drain():
        pltpu.make_async_copy(time_tab.at[pl.ds(0, HB), :], time_v,
                              sem).wait()
        pltpu.make_async_copy(time_tab.at[pl.ds(0, HB), :], rel_v,
                              sem).wait()

    def combine(half):
        def comb_body(i, carry):
            fi = jnp.full((L,), half * HB, jnp.int32) + i
            f = plsc.load_gather(fac_v, [fi])
            ivec = jnp.full((L,), half * HB, jnp.int32) + i
            for c in range(D // L):
                sl = pl.ds(c * L, L)
                v = time_v[i, sl] * f + rel_v[i, sl]
                plsc.store_scatter(combT_v, [jvecs[c], ivec], v)
            return carry

        lax.fori_loop(0, HB, comb_body, 0)

    rt_drain()
    combine(0)
    rt_issue(1)
    rt_drain()
    combine(1)
    pltpu.sync_copy(combT_v, outT_comb.at[:, pl.ds(base, BPW)])


@jax.jit
def _run(entity_indices, relation_indices, timestamp_indices, entity_table,
         relation_table, time_table, neg_decay):
    mesh = plsc.VectorSubcoreMesh(core_axis_name="c", subcore_axis_name="s",
                                  num_cores=NC, num_subcores=NS)
    comb_fn = functools.partial(
        pl.kernel,
        out_type=[jax.ShapeDtypeStruct((D, B), jnp.float32)],
        mesh=mesh,
        scratch_types=[
            pltpu.VMEM((BPW,), jnp.int32),
            pltpu.VMEM((BPW,), jnp.int32),
            pltpu.VMEM((BPW,), jnp.float32),
            pltpu.VMEM((L,), jnp.float32),
            pltpu.VMEM((HB, D), jnp.float32),
            pltpu.VMEM((HB, D), jnp.float32),
            pltpu.VMEM((D, BPW), jnp.float32),
            pltpu.SemaphoreType.DMA,
        ],
        compiler_params=_PARAMS,
        name="kg_comb_sc",
    )(_comb_kernel)
    ent_fn = functools.partial(
        pl.kernel,
        out_type=[jax.ShapeDtypeStruct((D, B), jnp.float32)],
        mesh=mesh,
        scratch_types=[
            pltpu.VMEM((BPW,), jnp.int32),
            pltpu.VMEM((BPW, D), jnp.float32),
            pltpu.VMEM((D, BPW), jnp.float32),
            pltpu.SemaphoreType.DMA,
        ],
        compiler_params=_PARAMS,
        name="kg_ent_sc",
    )(_ent_kernel)
    [outT_comb] = comb_fn(relation_indices, timestamp_indices,
                          relation_table, time_table, neg_decay)
    [outT_ent] = ent_fn(entity_indices, entity_table)
    return outT_ent, outT_comb


def kernel(entity_indices, relation_indices, timestamp_indices, entity_table,
           relation_table, time_table, time_decay):
    ent_idx = entity_indices.astype(jnp.int32)
    rel_idx = relation_indices.astype(jnp.int32)
    ts = timestamp_indices.astype(jnp.int32)
    neg_decay = jnp.broadcast_to(-time_decay, (L,))
    outT_ent, outT_comb = _run(ent_idx, rel_idx, ts, entity_table,
                               relation_table, time_table, neg_decay)
    return (outT_ent.T, outT_comb.T)


# speedup vs baseline: 1.0571x; 1.0471x over previous
import functools

import jax
import jax.numpy as jnp
from jax import lax
from jax.experimental import pallas as pl
from jax.experimental.pallas import tpu as pltpu
from jax.experimental.pallas import tpu_sc as plsc

B = 16384
D = 64
NC = 2
NS = 16
NW = NC * NS
BPW = B // NW
HB = BPW // 2
L = 16


def _sc_kernel(ent_idx_hbm, rel_idx_hbm, ts_hbm, ent_tab, rel_tab, time_tab,
               decay_hbm, out_ent, out_comb,
               eidx_v, ridx_v, ts_v, fac_v, dec_v, ent_v, time_v, rel_v,
               sem_e, sem_rt):
    wid = lax.axis_index("s") * NC + lax.axis_index("c")
    base = wid * BPW

    pltpu.sync_copy(ent_idx_hbm.at[pl.ds(base, BPW)], eidx_v)
    pltpu.sync_copy(rel_idx_hbm.at[pl.ds(base, BPW)], ridx_v)
    pltpu.sync_copy(ts_hbm.at[pl.ds(base, BPW)], ts_v)
    pltpu.sync_copy(decay_hbm, dec_v)

    def ent_issue(half):
        def issue(k, carry):
            off = pl.multiple_of(half * HB + k * L, L)
            vec = eidx_v[pl.ds(off, L)]
            dst = pl.multiple_of(k * L, L)
            for l in range(L):
                r = vec[l]
                pltpu.async_copy(ent_tab.at[pl.ds(r, 1), :],
                                 ent_v.at[pl.ds(dst + l, 1), :], sem_e)
            return carry

        lax.fori_loop(0, HB // L, issue, 0)

    def rt_issue(half):
        def issue(k, carry):
            off = pl.multiple_of(half * HB + k * L, L)
            vec = ridx_v[pl.ds(off, L)]
            dst = pl.multiple_of(k * L, L)
            for l in range(L):
                r = vec[l]
                pltpu.async_copy(time_tab.at[pl.ds(r, 1), :],
                                 time_v.at[pl.ds(dst + l, 1), :], sem_rt)
                pltpu.async_copy(rel_tab.at[pl.ds(r, 1), :],
                                 rel_v.at[pl.ds(dst + l, 1), :], sem_rt)
            return carry

        lax.fori_loop(0, HB // L, issue, 0)

    def ent_drain_write(half):
        pltpu.make_async_copy(ent_tab.at[pl.ds(0, HB), :], ent_v, sem_e).wait()
        pltpu.sync_copy(ent_v, out_ent.at[pl.ds(base + half * HB, HB), :])

    def rt_drain(half):
        pltpu.make_async_copy(ent_tab.at[pl.ds(0, HB), :], time_v,
                              sem_rt).wait()
        pltpu.make_async_copy(ent_tab.at[pl.ds(0, HB), :], rel_v,
                              sem_rt).wait()

    def combine_write(half):
        def comb_body(i, carry):
            fi = jnp.full((L,), half * HB, jnp.int32) + i
            f = plsc.load_gather(fac_v, [fi])
            for c in range(D // L):
                sl = pl.ds(c * L, L)
                time_v[i, sl] = time_v[i, sl] * f + rel_v[i, sl]
            return carry

        lax.fori_loop(0, HB, comb_body, 0)
        pltpu.sync_copy(time_v, out_comb.at[pl.ds(base + half * HB, HB), :])

    ent_issue(0)
    rt_issue(0)

    neg_decay = dec_v[...]

    def fac_body(k, carry):
        off = pl.multiple_of(k * L, L)
        t = ts_v[pl.ds(off, L)].astype(jnp.float32)
        fac_v[pl.ds(off, L)] = jnp.exp(neg_decay * t)
        return carry

    lax.fori_loop(0, BPW // L, fac_body, 0)

    ent_drain_write(0)
    ent_issue(1)
    rt_drain(0)
    combine_write(0)
    rt_issue(1)
    ent_drain_write(1)
    rt_drain(1)
    combine_write(1)


@jax.jit
def _run(entity_indices, relation_indices, timestamp_indices, entity_table,
         relation_table, time_table, neg_decay):
    mesh = plsc.VectorSubcoreMesh(core_axis_name="c", subcore_axis_name="s",
                                  num_cores=NC, num_subcores=NS)
    f = functools.partial(
        pl.kernel,
        out_type=[
            jax.ShapeDtypeStruct((B, D), jnp.float32),
            jax.ShapeDtypeStruct((B, D), jnp.float32),
        ],
        mesh=mesh,
        scratch_types=[
            pltpu.VMEM((BPW,), jnp.int32),
            pltpu.VMEM((BPW,), jnp.int32),
            pltpu.VMEM((BPW,), jnp.int32),
            pltpu.VMEM((BPW,), jnp.float32),
            pltpu.VMEM((L,), jnp.float32),
            pltpu.VMEM((HB, D), jnp.float32),
            pltpu.VMEM((HB, D), jnp.float32),
            pltpu.VMEM((HB, D), jnp.float32),
            pltpu.SemaphoreType.DMA,
            pltpu.SemaphoreType.DMA,
        ],
        compiler_params=pltpu.CompilerParams(needs_layout_passes=False,
                                             use_tc_tiling_on_sc=True),
        name="kg_embeddings_sc",
    )(_sc_kernel)
    return f(entity_indices, relation_indices, timestamp_indices,
             entity_table, relation_table, time_table, neg_decay)


def kernel(entity_indices, relation_indices, timestamp_indices, entity_table,
           relation_table, time_table, time_decay):
    ent_idx = entity_indices.astype(jnp.int32)
    rel_idx = relation_indices.astype(jnp.int32)
    ts = timestamp_indices.astype(jnp.int32)
    neg_decay = jnp.broadcast_to(-time_decay, (L,))
    out_ent, out_comb = _run(ent_idx, rel_idx, ts, entity_table,
                             relation_table, time_table, neg_decay)
    return (out_ent, out_comb)
